# Initial kernel scaffold; baseline (speedup 1.0000x reference)
import functools

import jax
import jax.numpy as jnp
from jax import lax
from jax.experimental import pallas as pl
from jax.experimental.pallas import tpu as pltpu
from jax.experimental.pallas import tpu_sc as plsc

B, S = 4, 2048
H = 768
Q = H // 4
NFMT = 11
N = B * S
EPS = 1e-12

NC, NS = 2, 16
NW = NC * NS
PW = N // NW
C = 32
NCHUNK = PW // C


def _sc_gather_sum(tok, ordr, nmag, npre, ntop, nlow,
                   token_W, order_W, mag_W, pre_W, top_W, low_W):
  mesh = plsc.VectorSubcoreMesh(core_axis_name="c", subcore_axis_name="s")

  @functools.partial(
      pl.kernel, mesh=mesh,
      out_type=jax.ShapeDtypeStruct((N, H), jnp.float32),
      scratch_types=[
          pltpu.VMEM((PW,), jnp.int32),
          pltpu.VMEM((PW,), jnp.int32),
          pltpu.VMEM((PW,), jnp.int32),
          pltpu.VMEM((PW,), jnp.int32),
          pltpu.VMEM((PW,), jnp.int32),
          pltpu.VMEM((PW,), jnp.int32),
          pltpu.VMEM((C, H), jnp.float32),
          pltpu.VMEM((C, H), jnp.float32),
          pltpu.VMEM((C, Q), jnp.float32),
          pltpu.VMEM((C, Q), jnp.float32),
          pltpu.VMEM((C, Q), jnp.float32),
          pltpu.VMEM((C, Q), jnp.float32),
          pltpu.SemaphoreType.DMA,
      ])
  def k(tok_h, ord_h, mag_h, pre_h, top_h, low_h,
        tw_h, ow_h, mw_h, pw_h, tpw_h, lw_h, out_h,
        tok_v, ord_v, mag_v, pre_v, top_v, low_v,
        A, Bv, q0, q1, q2, q3, sem):
    wid = lax.axis_index("s") * NC + lax.axis_index("c")
    base = wid * PW
    pltpu.sync_copy(tok_h.at[pl.ds(base, PW)], tok_v)
    pltpu.sync_copy(ord_h.at[pl.ds(base, PW)], ord_v)
    pltpu.sync_copy(mag_h.at[pl.ds(base, PW)], mag_v)
    pltpu.sync_copy(pre_h.at[pl.ds(base, PW)], pre_v)
    pltpu.sync_copy(top_h.at[pl.ds(base, PW)], top_v)
    pltpu.sync_copy(low_h.at[pl.ds(base, PW)], low_v)

    for chunk in range(NCHUNK):
      o = chunk * C
      cps = [
          pltpu.async_copy(tw_h.at[tok_v.at[pl.ds(o, C)]], A, sem),
          pltpu.async_copy(ow_h.at[ord_v.at[pl.ds(o, C)]], Bv, sem),
          pltpu.async_copy(mw_h.at[mag_v.at[pl.ds(o, C)]], q0, sem),
          pltpu.async_copy(pw_h.at[pre_v.at[pl.ds(o, C)]], q1, sem),
          pltpu.async_copy(tpw_h.at[top_v.at[pl.ds(o, C)]], q2, sem),
          pltpu.async_copy(lw_h.at[low_v.at[pl.ds(o, C)]], q3, sem),
      ]
      for cp in cps:
        cp.wait()

      def body(t, carry):
        for j in range(H // 16):
          sl = pl.ds(j * 16, 16)
          qi = (j * 16) // Q
          qref = (q0, q1, q2, q3)[qi]
          qs = pl.ds(j * 16 - qi * Q, 16)
          A[t, sl] = A[t, sl] + Bv[t, sl] + qref[t, qs]
        return carry
      lax.fori_loop(0, C, body, 0)

      pltpu.sync_copy(A, out_h.at[pl.ds(base + o, C)])

  return k(tok, ordr, nmag, npre, ntop, nlow,
           token_W, order_W, mag_W, pre_W, top_W, low_W)


def _tc_format_ln(G, fv, fw, gamma, beta):
  R = 256

  def body(g_ref, fv_ref, fw_ref, gam_ref, bet_ref, out_ref):
    f = lax.dot_general(fv_ref[...], fw_ref[...],
                        (((1,), (1,)), ((), ())),
                        preferred_element_type=jnp.float32)
    x = g_ref[...] + f
    mean = jnp.mean(x, axis=-1, keepdims=True)
    xc = x - mean
    var = jnp.mean(xc * xc, axis=-1, keepdims=True)
    y = xc * lax.rsqrt(var + EPS)
    out_ref[...] = y * gam_ref[...] + bet_ref[...]

  return pl.pallas_call(
      body,
      grid=(N // R,),
      in_specs=[
          pl.BlockSpec((R, H), lambda i: (i, 0)),
          pl.BlockSpec((R, NFMT), lambda i: (i, 0)),
          pl.BlockSpec((H, NFMT), lambda i: (0, 0)),
          pl.BlockSpec((1, H), lambda i: (0, 0)),
          pl.BlockSpec((1, H), lambda i: (0, 0)),
      ],
      out_specs=pl.BlockSpec((R, H), lambda i: (i, 0)),
      out_shape=jax.ShapeDtypeStruct((N, H), jnp.float32),
  )(G, fv, fw, gamma, beta)


def kernel(token_id, num_mag, num_pre, num_top, num_low, order, format_vec,
           token_W, mag_W, pre_W, top_W, low_W, order_W, format_W,
           ln_gamma, ln_beta):
  tok = token_id.reshape(N).astype(jnp.int32)
  ordr = order.reshape(N).astype(jnp.int32)
  nmag = num_mag.reshape(N).astype(jnp.int32)
  npre = num_pre.reshape(N).astype(jnp.int32)
  ntop = num_top.reshape(N).astype(jnp.int32)
  nlow = num_low.reshape(N).astype(jnp.int32)

  G = _sc_gather_sum(tok, ordr, nmag, npre, ntop, nlow,
                     token_W, order_W, mag_W, pre_W, top_W, low_W)

  fv = format_vec.reshape(N, NFMT)
  out = _tc_format_ln(G, fv, format_W,
                      ln_gamma.reshape(1, H), ln_beta.reshape(1, H))
  return out.reshape(B, S, H)


# speedup vs baseline: 1.6419x; 1.6419x over previous
import functools

import jax
import jax.numpy as jnp
from jax import lax
from jax.experimental import pallas as pl
from jax.experimental.pallas import tpu as pltpu
from jax.experimental.pallas import tpu_sc as plsc

B, S = 4, 2048
H = 768
Q = H // 4
QP = 256
NFMT = 11
N = B * S
EPS = 1e-12

NC, NS = 2, 16
NW = NC * NS
PW = N // NW
C = 32
NCHUNK = PW // C


def _sc_gather_sum(tok, ordr, nmag, npre, ntop, nlow,
                   token_W, order_W, mag_W, pre_W, top_W, low_W):
  mesh = plsc.VectorSubcoreMesh(core_axis_name="c", subcore_axis_name="s")

  @functools.partial(
      pl.kernel, mesh=mesh,
      out_type=jax.ShapeDtypeStruct((N, H), jnp.float32),
      scratch_types=[
          pltpu.VMEM((PW,), jnp.int32),
          pltpu.VMEM((PW,), jnp.int32),
          pltpu.VMEM((PW,), jnp.int32),
          pltpu.VMEM((PW,), jnp.int32),
          pltpu.VMEM((PW,), jnp.int32),
          pltpu.VMEM((PW,), jnp.int32),
          pltpu.VMEM((C, H), jnp.float32),
          pltpu.VMEM((C, H), jnp.float32),
          pltpu.VMEM((C, QP), jnp.float32),
          pltpu.VMEM((C, QP), jnp.float32),
          pltpu.VMEM((C, QP), jnp.float32),
          pltpu.VMEM((C, QP), jnp.float32),
          pltpu.SemaphoreType.DMA,
      ])
  def k(tok_h, ord_h, mag_h, pre_h, top_h, low_h,
        tw_h, ow_h, mw_h, pw_h, tpw_h, lw_h, out_h,
        tok_v, ord_v, mag_v, pre_v, top_v, low_v,
        A, Bv, q0, q1, q2, q3, sem):
    wid = lax.axis_index("s") * NC + lax.axis_index("c")
    base = wid * PW
    pltpu.sync_copy(tok_h.at[pl.ds(base, PW)], tok_v)
    pltpu.sync_copy(ord_h.at[pl.ds(base, PW)], ord_v)
    pltpu.sync_copy(mag_h.at[pl.ds(base, PW)], mag_v)
    pltpu.sync_copy(pre_h.at[pl.ds(base, PW)], pre_v)
    pltpu.sync_copy(top_h.at[pl.ds(base, PW)], top_v)
    pltpu.sync_copy(low_h.at[pl.ds(base, PW)], low_v)

    for chunk in range(NCHUNK):
      o = chunk * C
      cps = [
          pltpu.async_copy(tw_h.at[tok_v.at[pl.ds(o, C)]], A, sem),
          pltpu.async_copy(ow_h.at[ord_v.at[pl.ds(o, C)]], Bv, sem),
          pltpu.async_copy(mw_h.at[mag_v.at[pl.ds(o, C)]], q0, sem),
          pltpu.async_copy(pw_h.at[pre_v.at[pl.ds(o, C)]], q1, sem),
          pltpu.async_copy(tpw_h.at[top_v.at[pl.ds(o, C)]], q2, sem),
          pltpu.async_copy(lw_h.at[low_v.at[pl.ds(o, C)]], q3, sem),
      ]
      for cp in cps:
        cp.wait()

      def body(t, carry):
        for j in range(H // 16):
          sl = pl.ds(j * 16, 16)
          qi = (j * 16) // Q
          qref = (q0, q1, q2, q3)[qi]
          qs = pl.ds(j * 16 - qi * Q, 16)
          A[t, sl] = A[t, sl] + Bv[t, sl] + qref[t, qs]
        return carry
      lax.fori_loop(0, C, body, 0)

      pltpu.sync_copy(A, out_h.at[pl.ds(base + o, C)])

  return k(tok, ordr, nmag, npre, ntop, nlow,
           token_W, order_W, mag_W, pre_W, top_W, low_W)


def _tc_format_ln(G, fv, fw, gamma, beta):
  R = 256

  def body(g_ref, fv_ref, fw_ref, gam_ref, bet_ref, out_ref):
    f = lax.dot_general(fv_ref[...], fw_ref[...],
                        (((1,), (1,)), ((), ())),
                        preferred_element_type=jnp.float32)
    x = g_ref[...] + f
    mean = jnp.mean(x, axis=-1, keepdims=True)
    xc = x - mean
    var = jnp.mean(xc * xc, axis=-1, keepdims=True)
    y = xc * lax.rsqrt(var + EPS)
    out_ref[...] = y * gam_ref[...] + bet_ref[...]

  return pl.pallas_call(
      body,
      grid=(N // R,),
      in_specs=[
          pl.BlockSpec((R, H), lambda i: (i, 0)),
          pl.BlockSpec((R, NFMT), lambda i: (i, 0)),
          pl.BlockSpec((H, NFMT), lambda i: (0, 0)),
          pl.BlockSpec((1, H), lambda i: (0, 0)),
          pl.BlockSpec((1, H), lambda i: (0, 0)),
      ],
      out_specs=pl.BlockSpec((R, H), lambda i: (i, 0)),
      out_shape=jax.ShapeDtypeStruct((N, H), jnp.float32),
  )(G, fv, fw, gamma, beta)


def kernel(token_id, num_mag, num_pre, num_top, num_low, order, format_vec,
           token_W, mag_W, pre_W, top_W, low_W, order_W, format_W,
           ln_gamma, ln_beta):
  pad = ((0, 0), (0, QP - Q))
  mag_W = jnp.pad(mag_W, pad)
  pre_W = jnp.pad(pre_W, pad)
  top_W = jnp.pad(top_W, pad)
  low_W = jnp.pad(low_W, pad)

  tok = token_id.reshape(N).astype(jnp.int32)
  ordr = order.reshape(N).astype(jnp.int32)
  nmag = num_mag.reshape(N).astype(jnp.int32)
  npre = num_pre.reshape(N).astype(jnp.int32)
  ntop = num_top.reshape(N).astype(jnp.int32)
  nlow = num_low.reshape(N).astype(jnp.int32)

  G = _sc_gather_sum(tok, ordr, nmag, npre, ntop, nlow,
                     token_W, order_W, mag_W, pre_W, top_W, low_W)

  fv = format_vec.reshape(N, NFMT)
  out = _tc_format_ln(G, fv, format_W,
                      ln_gamma.reshape(1, H), ln_beta.reshape(1, H))
  return out.reshape(B, S, H)


# speedup vs baseline: 2.7989x; 1.7046x over previous
import functools

import jax
import jax.numpy as jnp
from jax import lax
from jax.experimental import pallas as pl
from jax.experimental.pallas import tpu as pltpu
from jax.experimental.pallas import tpu_sc as plsc

B, S = 4, 2048
H = 768
Q = H // 4
QP = 256
NUMV = 12
NFMT = 11
N = B * S
EPS = 1e-12

NC, NS = 2, 16
NW = NC * NS
PW = N // NW
C = 32
NCHUNK = PW // C


def _sc_gather_sum(tok, ordr, token_W, order_W):
  mesh = plsc.VectorSubcoreMesh(core_axis_name="c", subcore_axis_name="s")

  @functools.partial(
      pl.kernel, mesh=mesh,
      out_type=jax.ShapeDtypeStruct((N, H), jnp.float32),
      scratch_types=[
          pltpu.VMEM((PW,), jnp.int32),
          pltpu.VMEM((PW,), jnp.int32),
          pltpu.VMEM((C, H), jnp.float32),
          pltpu.VMEM((C, H), jnp.float32),
          pltpu.SemaphoreType.DMA,
      ])
  def k(tok_h, ord_h, tw_h, ow_h, out_h,
        tok_v, ord_v, A, Bv, sem):
    wid = lax.axis_index("s") * NC + lax.axis_index("c")
    base = wid * PW
    pltpu.sync_copy(tok_h.at[pl.ds(base, PW)], tok_v)
    pltpu.sync_copy(ord_h.at[pl.ds(base, PW)], ord_v)

    for chunk in range(NCHUNK):
      o = chunk * C
      cps = [
          pltpu.async_copy(tw_h.at[tok_v.at[pl.ds(o, C)]], A, sem),
          pltpu.async_copy(ow_h.at[ord_v.at[pl.ds(o, C)]], Bv, sem),
      ]
      for cp in cps:
        cp.wait()

      def body(t, carry):
        for j in range(H // 16):
          sl = pl.ds(j * 16, 16)
          A[t, sl] = A[t, sl] + Bv[t, sl]
        return carry
      lax.fori_loop(0, C, body, 0)

      pltpu.sync_copy(A, out_h.at[pl.ds(base + o, C)])

  return k(tok, ordr, token_W, order_W)


def _tc_format_ln(G, fv, fw, nidx, nW, gamma, beta):
  R = 256

  def body(g_ref, fv_ref, fw_ref, nidx_ref, nw_ref, gam_ref, bet_ref,
           out_ref):
    f = lax.dot_general(fv_ref[...], fw_ref[...],
                        (((1,), (1,)), ((), ())),
                        preferred_element_type=jnp.float32)
    cid = nidx_ref[...]
    iota12 = lax.broadcasted_iota(jnp.int32, (R, NUMV), 1)
    for q in range(4):
      oh = (cid[:, q:q + 1] == iota12).astype(jnp.float32)
      f = f + lax.dot_general(oh, nw_ref[pl.ds(q * NUMV, NUMV), :],
                              (((1,), (0,)), ((), ())),
                              preferred_element_type=jnp.float32)
    x = g_ref[...] + f
    mean = jnp.mean(x, axis=-1, keepdims=True)
    xc = x - mean
    var = jnp.mean(xc * xc, axis=-1, keepdims=True)
    y = xc * lax.rsqrt(var + EPS)
    out_ref[...] = y * gam_ref[...] + bet_ref[...]

  return pl.pallas_call(
      body,
      grid=(N // R,),
      in_specs=[
          pl.BlockSpec((R, H), lambda i: (i, 0)),
          pl.BlockSpec((R, NFMT), lambda i: (i, 0)),
          pl.BlockSpec((H, NFMT), lambda i: (0, 0)),
          pl.BlockSpec((R, 4), lambda i: (i, 0)),
          pl.BlockSpec((4 * NUMV, H), lambda i: (0, 0)),
          pl.BlockSpec((1, H), lambda i: (0, 0)),
          pl.BlockSpec((1, H), lambda i: (0, 0)),
      ],
      out_specs=pl.BlockSpec((R, H), lambda i: (i, 0)),
      out_shape=jax.ShapeDtypeStruct((N, H), jnp.float32),
  )(G, fv, fw, nidx, nW, gamma, beta)


def kernel(token_id, num_mag, num_pre, num_top, num_low, order, format_vec,
           token_W, mag_W, pre_W, top_W, low_W, order_W, format_W,
           ln_gamma, ln_beta):
  tok = token_id.reshape(N).astype(jnp.int32)
  ordr = order.reshape(N).astype(jnp.int32)

  G = _sc_gather_sum(tok, ordr, token_W, order_W)

  nidx = jnp.stack([num_mag.reshape(N), num_pre.reshape(N),
                    num_top.reshape(N), num_low.reshape(N)],
                   axis=1).astype(jnp.int32)
  nW = jnp.concatenate(
      [jnp.pad(w, ((0, 0), (q * Q, H - (q + 1) * Q)))
       for q, w in enumerate((mag_W, pre_W, top_W, low_W))], axis=0)

  fv = format_vec.reshape(N, NFMT)
  out = _tc_format_ln(G, fv, format_W, nidx, nW,
                      ln_gamma.reshape(1, H), ln_beta.reshape(1, H))
  return out.reshape(B, S, H)


# speedup vs baseline: 2.8121x; 1.0047x over previous
import functools

import jax
import jax.numpy as jnp
from jax import lax
from jax.experimental import pallas as pl
from jax.experimental.pallas import tpu as pltpu
from jax.experimental.pallas import tpu_sc as plsc

B, S = 4, 2048
H = 768
Q = H // 4
NUMV = 12
NFMT = 11
N = B * S
EPS = 1e-12

NC, NS = 2, 16
NW = NC * NS
C = 32
NH = 2
NT = N // NH


def _sc_gather_sum(tok, ordr, token_W, order_W):
  nt = tok.shape[0]
  pw = nt // NW
  nchunk = pw // C
  mesh = plsc.VectorSubcoreMesh(core_axis_name="c", subcore_axis_name="s")

  @functools.partial(
      pl.kernel, mesh=mesh,
      out_type=jax.ShapeDtypeStruct((nt, H), jnp.float32),
      scratch_types=[
          pltpu.VMEM((pw,), jnp.int32),
          pltpu.VMEM((pw,), jnp.int32),
          pltpu.VMEM((C, H), jnp.float32),
          pltpu.VMEM((C, H), jnp.float32),
          pltpu.SemaphoreType.DMA,
      ])
  def k(tok_h, ord_h, tw_h, ow_h, out_h, tok_v, ord_v, A, Bv, sem):
    wid = lax.axis_index("s") * NC + lax.axis_index("c")
    base = wid * pw
    pltpu.sync_copy(tok_h.at[pl.ds(base, pw)], tok_v)
    pltpu.sync_copy(ord_h.at[pl.ds(base, pw)], ord_v)

    for chunk in range(nchunk):
      o = chunk * C
      cps = [
          pltpu.async_copy(tw_h.at[tok_v.at[pl.ds(o, C)]], A, sem),
          pltpu.async_copy(ow_h.at[ord_v.at[pl.ds(o, C)]], Bv, sem),
      ]
      for cp in cps:
        cp.wait()

      def body(t, carry):
        for j in range(H // 16):
          sl = pl.ds(j * 16, 16)
          A[t, sl] = A[t, sl] + Bv[t, sl]
        return carry
      lax.fori_loop(0, C, body, 0)

      pltpu.sync_copy(A, out_h.at[pl.ds(base + o, C)])

  return k(tok, ordr, token_W, order_W)


def _tc_format_ln(G, fv, fw, i0, i1, i2, i3, w0, w1, w2, w3, gamma, beta,
                  row0, buf):
  R = 256
  nt = G.shape[0]

  def body(*refs):
    (g_ref, fv_ref, fw_ref, i0_ref, i1_ref, i2_ref, i3_ref,
     w0_ref, w1_ref, w2_ref, w3_ref, gam_ref, bet_ref) = refs[:13]
    out_ref = refs[-1]
    f = lax.dot_general(fv_ref[...], fw_ref[...],
                        (((1,), (1,)), ((), ())),
                        preferred_element_type=jnp.float32)
    iota12 = lax.broadcasted_iota(jnp.int32, (R, NUMV), 1)
    qs = []
    for i_ref, w_ref in ((i0_ref, w0_ref), (i1_ref, w1_ref),
                         (i2_ref, w2_ref), (i3_ref, w3_ref)):
      oh = (i_ref[...] == iota12).astype(jnp.float32)
      qs.append(lax.dot_general(oh, w_ref[...], (((1,), (0,)), ((), ())),
                                preferred_element_type=jnp.float32))
    x = g_ref[...] + f + jnp.concatenate(qs, axis=1)
    mean = jnp.mean(x, axis=-1, keepdims=True)
    xc = x - mean
    var = jnp.mean(xc * xc, axis=-1, keepdims=True)
    y = xc * lax.rsqrt(var + EPS)
    out_ref[...] = y * gam_ref[...] + bet_ref[...]

  in_specs = [
      pl.BlockSpec((R, H), lambda i: (i, 0)),
      pl.BlockSpec((R, NFMT), lambda i: (i, 0)),
      pl.BlockSpec((H, NFMT), lambda i: (0, 0)),
      pl.BlockSpec((R, 1), lambda i: (i, 0)),
      pl.BlockSpec((R, 1), lambda i: (i, 0)),
      pl.BlockSpec((R, 1), lambda i: (i, 0)),
      pl.BlockSpec((R, 1), lambda i: (i, 0)),
      pl.BlockSpec((NUMV, Q), lambda i: (0, 0)),
      pl.BlockSpec((NUMV, Q), lambda i: (0, 0)),
      pl.BlockSpec((NUMV, Q), lambda i: (0, 0)),
      pl.BlockSpec((NUMV, Q), lambda i: (0, 0)),
      pl.BlockSpec((1, H), lambda i: (0, 0)),
      pl.BlockSpec((1, H), lambda i: (0, 0)),
  ]
  args = [G, fv, fw, i0, i1, i2, i3, w0, w1, w2, w3, gamma, beta]
  kwargs = {}
  if buf is not None:
    in_specs.append(pl.BlockSpec(memory_space=pl.ANY))
    args.append(buf)
    kwargs["input_output_aliases"] = {len(args) - 1: 0}
  blk0 = row0 // R

  return pl.pallas_call(
      body,
      grid=(nt // R,),
      in_specs=in_specs,
      out_specs=pl.BlockSpec((R, H), lambda i: (i + blk0, 0)),
      out_shape=jax.ShapeDtypeStruct((N, H), jnp.float32),
      **kwargs,
  )(*args)


def kernel(token_id, num_mag, num_pre, num_top, num_low, order, format_vec,
           token_W, mag_W, pre_W, top_W, low_W, order_W, format_W,
           ln_gamma, ln_beta):
  tok = token_id.reshape(N).astype(jnp.int32)
  ordr = order.reshape(N).astype(jnp.int32)
  i0 = num_mag.reshape(N, 1).astype(jnp.int32)
  i1 = num_pre.reshape(N, 1).astype(jnp.int32)
  i2 = num_top.reshape(N, 1).astype(jnp.int32)
  i3 = num_low.reshape(N, 1).astype(jnp.int32)
  fv = format_vec.reshape(N, NFMT)
  gamma = ln_gamma.reshape(1, H)
  beta = ln_beta.reshape(1, H)

  Gs = [_sc_gather_sum(tok[h * NT:(h + 1) * NT], ordr[h * NT:(h + 1) * NT],
                       token_W, order_W) for h in range(NH)]

  buf = None
  for h in range(NH):
    sl = slice(h * NT, (h + 1) * NT)
    buf = _tc_format_ln(Gs[h], fv[sl], format_W,
                        i0[sl], i1[sl], i2[sl], i3[sl],
                        mag_W, pre_W, top_W, low_W, gamma, beta,
                        h * NT, buf)
  return buf.reshape(B, S, H)


# speedup vs baseline: 3.2949x; 1.1717x over previous
import functools

import jax
import jax.numpy as jnp
from jax import lax
from jax.experimental import pallas as pl
from jax.experimental.pallas import tpu as pltpu
from jax.experimental.pallas import tpu_sc as plsc

B, S = 4, 2048
H = 768
Q = H // 4
NUMV = 12
NFMT = 11
N = B * S
EPS = 1e-12

NC, NS = 2, 16
NW = NC * NS
PW = N // NW
C = 32
NCHUNK = PW // C


def _sc_gather_sum(tok, ordr, token_W, order_W):
  mesh = plsc.VectorSubcoreMesh(core_axis_name="c", subcore_axis_name="s")

  @functools.partial(
      pl.kernel, mesh=mesh,
      out_type=jax.ShapeDtypeStruct((N, H), jnp.float32),
      scratch_types=[
          pltpu.VMEM((PW,), jnp.int32),
          pltpu.VMEM((PW,), jnp.int32),
          pltpu.VMEM((C, H), jnp.float32),
          pltpu.VMEM((C, H), jnp.float32),
          pltpu.VMEM((C, H), jnp.float32),
          pltpu.VMEM((C, H), jnp.float32),
          pltpu.SemaphoreType.DMA,
          pltpu.SemaphoreType.DMA,
          pltpu.SemaphoreType.DMA,
          pltpu.SemaphoreType.DMA,
      ])
  def k(tok_h, ord_h, tw_h, ow_h, out_h, tok_v, ord_v,
        A0, B0, A1, B1, sg0, sg1, sw0, sw1):
    wid = lax.axis_index("s") * NC + lax.axis_index("c")
    base = wid * PW
    pltpu.sync_copy(tok_h.at[pl.ds(base, PW)], tok_v)
    pltpu.sync_copy(ord_h.at[pl.ds(base, PW)], ord_v)

    Ab, Bb = (A0, A1), (B0, B1)
    sg, sw = (sg0, sg1), (sw0, sw1)
    gh = [None, None]
    wb = [None, None]

    for i in range(NCHUNK + 1):
      s = i % 2
      if i < NCHUNK:
        o = i * C
        if wb[s] is not None:
          wb[s].wait()
          wb[s] = None
        gh[s] = (
            pltpu.async_copy(tw_h.at[tok_v.at[pl.ds(o, C)]], Ab[s], sg[s]),
            pltpu.async_copy(ow_h.at[ord_v.at[pl.ds(o, C)]], Bb[s], sg[s]),
        )
      if i > 0:
        sp = (i - 1) % 2
        for cp in gh[sp]:
          cp.wait()
        A, Bv = Ab[sp], Bb[sp]

        def body(t, carry):
          for j in range(H // 16):
            sl = pl.ds(j * 16, 16)
            A[t, sl] = A[t, sl] + Bv[t, sl]
          return carry
        lax.fori_loop(0, C, body, 0)

        wb[sp] = pltpu.async_copy(
            A, out_h.at[pl.ds(base + (i - 1) * C, C)], sw[sp])

    for s in (0, 1):
      if wb[s] is not None:
        wb[s].wait()

  return k(tok, ordr, token_W, order_W)


def _tc_format_ln(G, fv3, fw, nidx, w0, w1, w2, w3, gamma, beta):
  R = 256

  def body(g_ref, fv_ref, fw_ref, nidx_ref,
           w0_ref, w1_ref, w2_ref, w3_ref, gam_ref, bet_ref, out_ref):
    fv = fv_ref[...].reshape(R, NFMT)
    f = lax.dot_general(fv, fw_ref[...], (((1,), (1,)), ((), ())),
                        preferred_element_type=jnp.float32)
    cid = nidx_ref[...]
    iota12 = lax.broadcasted_iota(jnp.int32, (R, NUMV), 1)
    qs = []
    for q, w_ref in enumerate((w0_ref, w1_ref, w2_ref, w3_ref)):
      oh = (cid[:, q:q + 1] == iota12).astype(jnp.float32)
      qs.append(lax.dot_general(oh, w_ref[...], (((1,), (0,)), ((), ())),
                                preferred_element_type=jnp.float32))
    x = g_ref[...] + f + jnp.concatenate(qs, axis=1)
    mean = jnp.mean(x, axis=-1, keepdims=True)
    xc = x - mean
    var = jnp.mean(xc * xc, axis=-1, keepdims=True)
    y = xc * lax.rsqrt(var + EPS)
    out_ref[...] = y * gam_ref[...] + bet_ref[...]

  return pl.pallas_call(
      body,
      grid=(N // R,),
      in_specs=[
          pl.BlockSpec((R, H), lambda i: (i, 0)),
          pl.BlockSpec((1, R, NFMT), lambda i: (i, 0, 0)),
          pl.BlockSpec((H, NFMT), lambda i: (0, 0)),
          pl.BlockSpec((R, 4), lambda i: (i, 0)),
          pl.BlockSpec((NUMV, Q), lambda i: (0, 0)),
          pl.BlockSpec((NUMV, Q), lambda i: (0, 0)),
          pl.BlockSpec((NUMV, Q), lambda i: (0, 0)),
          pl.BlockSpec((NUMV, Q), lambda i: (0, 0)),
          pl.BlockSpec((1, H), lambda i: (0, 0)),
          pl.BlockSpec((1, H), lambda i: (0, 0)),
      ],
      out_specs=pl.BlockSpec((R, H), lambda i: (i, 0)),
      out_shape=jax.ShapeDtypeStruct((N, H), jnp.float32),
  )(G, fv3, fw, nidx, w0, w1, w2, w3, gamma, beta)


def kernel(token_id, num_mag, num_pre, num_top, num_low, order, format_vec,
           token_W, mag_W, pre_W, top_W, low_W, order_W, format_W,
           ln_gamma, ln_beta):
  tok = token_id.reshape(N).astype(jnp.int32)
  ordr = order.reshape(N).astype(jnp.int32)

  G = _sc_gather_sum(tok, ordr, token_W, order_W)

  nidx = jnp.stack([num_mag.reshape(N), num_pre.reshape(N),
                    num_top.reshape(N), num_low.reshape(N)],
                   axis=1).astype(jnp.int32)
  fv3 = format_vec.reshape(N // 256, 256, NFMT)
  out = _tc_format_ln(G, fv3, format_W, nidx, mag_W, pre_W, top_W, low_W,
                      ln_gamma.reshape(1, H), ln_beta.reshape(1, H))
  return out.reshape(B, S, H)


# speedup vs baseline: 4.6885x; 1.4229x over previous
import functools

import jax
import jax.numpy as jnp
from jax import lax
from jax.experimental import pallas as pl
from jax.experimental.pallas import tpu as pltpu
from jax.experimental.pallas import tpu_sc as plsc

B, S = 4, 2048
H = 768
Q = H // 4
NUMV = 12
MAXCELL = 256
NFMT = 11
N = B * S
EPS = 1e-12

NC, NS = 2, 16
NW = NC * NS
PW = N // NW
C = 64
NCHUNK = PW // C

R = 512
NBLK = N // R


def _sc_token_gather(tok, token_W):
  mesh = plsc.VectorSubcoreMesh(core_axis_name="c", subcore_axis_name="s")

  @functools.partial(
      pl.kernel, mesh=mesh,
      out_type=jax.ShapeDtypeStruct((N, H), jnp.float32),
      scratch_types=[
          pltpu.VMEM((PW,), jnp.int32),
          pltpu.VMEM((C, H), jnp.float32),
          pltpu.VMEM((C, H), jnp.float32),
          pltpu.SemaphoreType.DMA,
          pltpu.SemaphoreType.DMA,
          pltpu.SemaphoreType.DMA,
          pltpu.SemaphoreType.DMA,
      ])
  def k(tok_h, tw_h, out_h, tok_v, A0, A1, sg0, sg1, sw0, sw1):
    wid = lax.axis_index("s") * NC + lax.axis_index("c")
    base = wid * PW
    pltpu.sync_copy(tok_h.at[pl.ds(base, PW)], tok_v)

    Ab = (A0, A1)
    sg, sw = (sg0, sg1), (sw0, sw1)
    gh = [None, None]
    wb = [None, None]

    for i in range(NCHUNK + 1):
      s = i % 2
      if i < NCHUNK:
        if wb[s] is not None:
          wb[s].wait()
          wb[s] = None
        gh[s] = pltpu.async_copy(
            tw_h.at[tok_v.at[pl.ds(i * C, C)]], Ab[s], sg[s])
      if i > 0:
        sp = (i - 1) % 2
        gh[sp].wait()
        wb[sp] = pltpu.async_copy(
            Ab[sp], out_h.at[pl.ds(base + (i - 1) * C, C)], sw[sp])

    for s in (0, 1):
      if wb[s] is not None:
        wb[s].wait()

  return k(tok, token_W)


def _tc_dense_ln(G, fv3, fw, pidx, oW, nW, gamma, beta):
  def body(g_ref, fv_ref, fw_ref, pidx_ref, ow_ref, nw_ref,
           gam_ref, bet_ref, out_ref):
    fv = fv_ref[...].reshape(R, NFMT)
    f = lax.dot_general(fv, fw_ref[...], (((1,), (1,)), ((), ())),
                        preferred_element_type=jnp.float32)

    pk = pidx_ref[...].reshape(1, R)
    ordv = (pk >> 16) & 0xFF
    iota_o = lax.broadcasted_iota(jnp.int32, (MAXCELL, R), 0)
    oh_o = (iota_o == ordv).astype(jnp.float32)
    f = f + lax.dot_general(oh_o, ow_ref[...], (((0,), (0,)), ((), ())),
                            preferred_element_type=jnp.float32)
    iota_n = lax.broadcasted_iota(jnp.int32, (NUMV, R), 0)
    ohs = [(iota_n == ((pk >> (4 * q)) & 0xF)).astype(jnp.float32)
           for q in range(4)]
    oh_n = jnp.concatenate(ohs, axis=0)
    f = f + lax.dot_general(oh_n, nw_ref[...], (((0,), (0,)), ((), ())),
                            preferred_element_type=jnp.float32)

    x = g_ref[...] + f
    mean = jnp.mean(x, axis=-1, keepdims=True)
    xc = x - mean
    var = jnp.mean(xc * xc, axis=-1, keepdims=True)
    y = xc * lax.rsqrt(var + EPS)
    out_ref[...] = y * gam_ref[...] + bet_ref[...]

  return pl.pallas_call(
      body,
      grid=(NBLK,),
      in_specs=[
          pl.BlockSpec((R, H), lambda i: (i, 0)),
          pl.BlockSpec((1, R, NFMT), lambda i: (i, 0, 0)),
          pl.BlockSpec((H, NFMT), lambda i: (0, 0)),
          pl.BlockSpec((1, 1, R), lambda i: (i, 0, 0)),
          pl.BlockSpec((MAXCELL, H), lambda i: (0, 0)),
          pl.BlockSpec((4 * NUMV, H), lambda i: (0, 0)),
          pl.BlockSpec((1, H), lambda i: (0, 0)),
          pl.BlockSpec((1, H), lambda i: (0, 0)),
      ],
      out_specs=pl.BlockSpec((R, H), lambda i: (i, 0)),
      out_shape=jax.ShapeDtypeStruct((N, H), jnp.float32),
  )(G, fv3, fw, pidx, oW, nW, gamma, beta)


def kernel(token_id, num_mag, num_pre, num_top, num_low, order, format_vec,
           token_W, mag_W, pre_W, top_W, low_W, order_W, format_W,
           ln_gamma, ln_beta):
  tok = token_id.reshape(N).astype(jnp.int32)

  G = _sc_token_gather(tok, token_W)

  packed = (num_mag.astype(jnp.int32)
            | (num_pre.astype(jnp.int32) << 4)
            | (num_top.astype(jnp.int32) << 8)
            | (num_low.astype(jnp.int32) << 12)
            | (order.astype(jnp.int32) << 16))
  pidx = packed.reshape(NBLK, 1, R)

  nW = jnp.concatenate(
      [jnp.pad(w, ((0, 0), (q * Q, H - (q + 1) * Q)))
       for q, w in enumerate((mag_W, pre_W, top_W, low_W))], axis=0)

  fv3 = format_vec.reshape(NBLK, R, NFMT)
  out = _tc_dense_ln(G, fv3, format_W, pidx, order_W, nW,
                     ln_gamma.reshape(1, H), ln_beta.reshape(1, H))
  return out.reshape(B, S, H)


# speedup vs baseline: 4.9849x; 1.0632x over previous
import functools

import jax
import jax.numpy as jnp
from jax import lax
from jax.experimental import pallas as pl
from jax.experimental.pallas import tpu as pltpu
from jax.experimental.pallas import tpu_sc as plsc

B, S = 4, 2048
H = 768
Q = H // 4
NUMV = 12
MAXCELL = 256
NFMT = 11
N = B * S
EPS = 1e-12

NC, NS = 2, 16
NW = NC * NS
PW = N // NW
C = 64
NCHUNK = PW // C

R = 1024
NBLK = N // R
SBLK = S // R


def _sc_token_gather(tok, token_W):
  mesh = plsc.VectorSubcoreMesh(core_axis_name="c", subcore_axis_name="s")

  @functools.partial(
      pl.kernel, mesh=mesh,
      out_type=jax.ShapeDtypeStruct((N, H), jnp.float32),
      scratch_types=[
          pltpu.VMEM((PW,), jnp.int32),
          pltpu.VMEM((C, H), jnp.float32),
          pltpu.VMEM((C, H), jnp.float32),
          pltpu.SemaphoreType.DMA,
          pltpu.SemaphoreType.DMA,
          pltpu.SemaphoreType.DMA,
          pltpu.SemaphoreType.DMA,
      ])
  def k(tok_h, tw_h, out_h, tok_v, A0, A1, sg0, sg1, sw0, sw1):
    wid = lax.axis_index("s") * NC + lax.axis_index("c")
    base = wid * PW
    pltpu.sync_copy(tok_h.at[pl.ds(base, PW)], tok_v)

    Ab = (A0, A1)
    sg, sw = (sg0, sg1), (sw0, sw1)
    gh = [None, None]
    wb = [None, None]

    for i in range(NCHUNK + 1):
      s = i % 2
      if i < NCHUNK:
        if wb[s] is not None:
          wb[s].wait()
          wb[s] = None
        gh[s] = pltpu.async_copy(
            tw_h.at[tok_v.at[pl.ds(i * C, C)]], Ab[s], sg[s])
      if i > 0:
        sp = (i - 1) % 2
        gh[sp].wait()
        wb[sp] = pltpu.async_copy(
            Ab[sp], out_h.at[pl.ds(base + (i - 1) * C, C)], sw[sp])

    for s in (0, 1):
      if wb[s] is not None:
        wb[s].wait()

  return k(tok, token_W)


def _tc_dense_ln(G, fv3, fw, pidx, oW, nW, gamma, beta):
  def body(g_ref, fv_ref, fw_ref, pidx_ref, ow_ref, nw_ref,
           gam_ref, bet_ref, out_ref):
    fv = fv_ref[...].reshape(R, NFMT)
    f = lax.dot_general(fv, fw_ref[...], (((1,), (1,)), ((), ())),
                        preferred_element_type=jnp.float32)

    pk = pidx_ref[...].reshape(1, R)
    ordv = (pk >> 16) & 0xFF
    iota_o = lax.broadcasted_iota(jnp.int32, (MAXCELL, R), 0)
    oh_o = (iota_o == ordv).astype(jnp.float32)
    f = f + lax.dot_general(oh_o, ow_ref[...], (((0,), (0,)), ((), ())),
                            preferred_element_type=jnp.float32)
    iota_n = lax.broadcasted_iota(jnp.int32, (NUMV, R), 0)
    ohs = [(iota_n == ((pk >> (4 * q)) & 0xF)).astype(jnp.float32)
           for q in range(4)]
    oh_n = jnp.concatenate(ohs, axis=0)
    f = f + lax.dot_general(oh_n, nw_ref[...], (((0,), (0,)), ((), ())),
                            preferred_element_type=jnp.float32)

    x = g_ref[...] + f
    mean = jnp.mean(x, axis=-1, keepdims=True)
    xc = x - mean
    var = jnp.mean(xc * xc, axis=-1, keepdims=True)
    y = xc * lax.rsqrt(var + EPS)
    out_ref[...] = y * gam_ref[...] + bet_ref[...]

  return pl.pallas_call(
      body,
      grid=(NBLK,),
      in_specs=[
          pl.BlockSpec((R, H), lambda i: (i, 0)),
          pl.BlockSpec((1, R, NFMT), lambda i: (i // SBLK, i % SBLK, 0)),
          pl.BlockSpec((H, NFMT), lambda i: (0, 0)),
          pl.BlockSpec((1, 1, R), lambda i: (i, 0, 0)),
          pl.BlockSpec((MAXCELL, H), lambda i: (0, 0)),
          pl.BlockSpec((4 * NUMV, H), lambda i: (0, 0)),
          pl.BlockSpec((1, H), lambda i: (0, 0)),
          pl.BlockSpec((1, H), lambda i: (0, 0)),
      ],
      out_specs=pl.BlockSpec((R, H), lambda i: (i, 0)),
      out_shape=jax.ShapeDtypeStruct((N, H), jnp.float32),
  )(G, fv3, fw, pidx, oW, nW, gamma, beta)


def kernel(token_id, num_mag, num_pre, num_top, num_low, order, format_vec,
           token_W, mag_W, pre_W, top_W, low_W, order_W, format_W,
           ln_gamma, ln_beta):
  tok = token_id.reshape(N).astype(jnp.int32)

  G = _sc_token_gather(tok, token_W)

  packed = (num_mag.astype(jnp.int32)
            | (num_pre.astype(jnp.int32) << 4)
            | (num_top.astype(jnp.int32) << 8)
            | (num_low.astype(jnp.int32) << 12)
            | (order.astype(jnp.int32) << 16))
  pidx = packed.reshape(NBLK, 1, R)

  nW = jnp.concatenate(
      [jnp.pad(w, ((0, 0), (q * Q, H - (q + 1) * Q)))
       for q, w in enumerate((mag_W, pre_W, top_W, low_W))], axis=0)

  out = _tc_dense_ln(G, format_vec, format_W, pidx, order_W, nW,
                     ln_gamma.reshape(1, H), ln_beta.reshape(1, H))
  return out.reshape(B, S, H)
